# Initial kernel scaffold; baseline (speedup 1.0000x reference)
import jax
import jax.numpy as jnp
from jax.experimental import pallas as pl


def kernel(texts, table, W, b):
    raise NotImplementedError("write your pallas kernel here")


# speedup vs baseline: 2.3653x; 2.3653x over previous
import functools

import jax
import jax.numpy as jnp
from jax import lax
from jax.experimental import pallas as pl
from jax.experimental.pallas import tpu as pltpu
from jax.experimental.pallas import tpu_sc as plsc

_B = 16384
_L = 50
_E = 64
_EPS = 1e-10

_NC = 2
_NS = 16
_NW = _NC * _NS
_PB = _B // _NW
_CB = 8
_NST = _PB // _CB
_IPG = 100
_GPS = (_CB * _L) // _IPG


def _sc_body(texts2, table, sums, idx_buf, rows_buf, out_buf, gsem):
    c = lax.axis_index("c")
    s = lax.axis_index("s")
    wid = s * _NC + c
    e_base = wid * _PB
    q_base = wid * (_PB // 2)

    def stage(st, carry):
        e0 = e_base + st * _CB
        q0 = q_base + st * _GPS
        pltpu.sync_copy(texts2.at[pl.ds(q0, _GPS)], idx_buf)
        descs = []
        for j in range(_GPS):
            descs.append(
                pltpu.async_copy(
                    table.at[idx_buf.at[j]],
                    rows_buf.at[pl.ds(j * _IPG, _IPG)],
                    gsem,
                )
            )
        for d in descs:
            d.wait()
        for e in range(_CB):
            r0 = e * _L
            accs = tuple(
                rows_buf[r0, pl.ds(16 * k, 16)] for k in range(_E // 16)
            )

            def lbody(l, a):
                return tuple(
                    a[k] + rows_buf[r0 + l, pl.ds(16 * k, 16)]
                    for k in range(_E // 16)
                )

            accs = lax.fori_loop(1, _L, lbody, accs, unroll=7)
            for k in range(_E // 16):
                out_buf[e, pl.ds(16 * k, 16)] = accs[k]
        pltpu.sync_copy(out_buf, sums.at[pl.ds(e0, _CB)])
        return carry

    lax.fori_loop(0, _NST, stage, 0)


@functools.partial(jax.jit, static_argnames=())
def _sc_gather_sum(texts2, table):
    mesh = plsc.VectorSubcoreMesh(core_axis_name="c", subcore_axis_name="s")
    fn = pl.kernel(
        _sc_body,
        out_type=jax.ShapeDtypeStruct((_B, _E), jnp.float32),
        mesh=mesh,
        scratch_types=[
            pltpu.VMEM((_GPS, _IPG), jnp.int32),
            pltpu.VMEM((_CB * _L, _E), jnp.float32),
            pltpu.VMEM((_CB, _E), jnp.float32),
            pltpu.SemaphoreType.DMA,
        ],
        compiler_params=pltpu.CompilerParams(use_tc_tiling_on_sc=False),
    )
    return fn(texts2, table)


def _tc_body(sums_ref, texts_ref, wt_ref, b_ref, out_ref, mask_ref):
    cnt = jnp.sum((texts_ref[...] != 0).astype(jnp.float32), axis=1,
                  keepdims=True)
    s = cnt + _EPS
    avg = sums_ref[...] / s
    out_ref[...] = (
        jnp.dot(avg, wt_ref[...], preferred_element_type=jnp.float32)
        + b_ref[...]
    )
    mask_ref[...] = (s != 0).astype(jnp.int32)


def _tc_finish(sums, texts, wt, b2):
    bt = 2048
    grid = (_B // bt,)
    return pl.pallas_call(
        _tc_body,
        grid=grid,
        in_specs=[
            pl.BlockSpec((bt, _E), lambda i: (i, 0)),
            pl.BlockSpec((bt, _L), lambda i: (i, 0)),
            pl.BlockSpec((_E, _E), lambda i: (0, 0)),
            pl.BlockSpec((1, _E), lambda i: (0, 0)),
        ],
        out_specs=[
            pl.BlockSpec((bt, _E), lambda i: (i, 0)),
            pl.BlockSpec((bt, 1), lambda i: (i, 0)),
        ],
        out_shape=[
            jax.ShapeDtypeStruct((_B, _E), jnp.float32),
            jax.ShapeDtypeStruct((_B, 1), jnp.int32),
        ],
    )(sums, texts, wt, b2)


def kernel(texts, table, W, b):
    assert texts.shape == (_B, _L)
    texts2 = texts.reshape(_B // 2, 2 * _L)
    sums = _sc_gather_sum(texts2, table)
    out, mask = _tc_finish(sums, texts, W.T, b.reshape(1, _E))
    return (out, mask)


# speedup vs baseline: 2.7319x; 1.1550x over previous
import functools

import jax
import jax.numpy as jnp
from jax import lax
from jax.experimental import pallas as pl
from jax.experimental.pallas import tpu as pltpu
from jax.experimental.pallas import tpu_sc as plsc

_B = 16384
_L = 50
_E = 64
_EPS = 1e-10

_NC = 2
_NS = 16
_NW = _NC * _NS
_PB = _B // _NW
_CB = 8
_NST = _PB // _CB
_IPG = 100
_GPS = (_CB * _L) // _IPG
_NK = _E // 16


def _sc_body(texts2, table, sums, idx_buf, rows_buf, out_buf,
             isem0, isem1, gsem0, gsem1, osem0, osem1):
    isem = (isem0, isem1)
    gsem = (gsem0, gsem1)
    osem = (osem0, osem1)
    c = lax.axis_index("c")
    s = lax.axis_index("s")
    wid = s * _NC + c
    e_base = wid * _PB
    q_base = wid * (_PB // 2)

    def idx_copy(st, p):
        q0 = q_base + st * _GPS
        pltpu.async_copy(texts2.at[pl.ds(q0, _GPS)], idx_buf.at[p], isem[p])

    def wait_idx(p):
        pltpu.make_async_copy(texts2.at[pl.ds(0, _GPS)],
                              idx_buf.at[p], isem[p]).wait()

    def fire_gathers(p):
        for j in range(_GPS):
            pltpu.async_copy(
                table.at[idx_buf.at[p, j]],
                rows_buf.at[p, pl.ds(j * _IPG, _IPG)],
                gsem[p],
            )

    def wait_gathers(p):
        pltpu.make_async_copy(table.at[pl.ds(0, _GPS * _IPG)],
                              rows_buf.at[p], gsem[p]).wait()

    def accumulate(p):
        for e in range(_CB):
            r0 = e * _L
            accs = tuple(
                rows_buf[p, r0, pl.ds(16 * k, 16)] for k in range(_NK)
            )

            def lbody(l, a):
                return tuple(
                    a[k] + rows_buf[p, r0 + l, pl.ds(16 * k, 16)]
                    for k in range(_NK)
                )

            accs = lax.fori_loop(1, _L, lbody, accs, unroll=7)
            for k in range(_NK):
                out_buf[p, e, pl.ds(16 * k, 16)] = accs[k]

    def out_write(st, p):
        e0 = e_base + st * _CB
        pltpu.async_copy(out_buf.at[p], sums.at[pl.ds(e0, _CB)], osem[p])

    def wait_out(p):
        pltpu.make_async_copy(out_buf.at[p], sums.at[pl.ds(e_base, _CB)],
                              osem[p]).wait()

    idx_copy(0, 0)
    wait_idx(0)
    fire_gathers(0)
    idx_copy(1, 1)

    def outer(i, carry):
        st0 = i * 2
        for b in range(2):
            st = st0 + b
            p = b

            @pl.when(st + 1 < _NST)
            def _():
                wait_idx(1 - p)
                fire_gathers(1 - p)

            wait_gathers(p)

            @pl.when(st + 2 < _NST)
            def _():
                idx_copy(st + 2, p)

            @pl.when(st >= 2)
            def _():
                wait_out(p)

            accumulate(p)
            out_write(st, p)
        return carry

    lax.fori_loop(0, _NST // 2, outer, 0)
    wait_out(0)
    wait_out(1)


@functools.partial(jax.jit, static_argnames=())
def _sc_gather_sum(texts2, table):
    mesh = plsc.VectorSubcoreMesh(core_axis_name="c", subcore_axis_name="s")
    fn = pl.kernel(
        _sc_body,
        out_type=jax.ShapeDtypeStruct((_B, _E), jnp.float32),
        mesh=mesh,
        scratch_types=[
            pltpu.VMEM((2, _GPS, _IPG), jnp.int32),
            pltpu.VMEM((2, _CB * _L, _E), jnp.float32),
            pltpu.VMEM((2, _CB, _E), jnp.float32),
            pltpu.SemaphoreType.DMA,
            pltpu.SemaphoreType.DMA,
            pltpu.SemaphoreType.DMA,
            pltpu.SemaphoreType.DMA,
            pltpu.SemaphoreType.DMA,
            pltpu.SemaphoreType.DMA,
        ],
        compiler_params=pltpu.CompilerParams(use_tc_tiling_on_sc=False),
    )
    return fn(texts2, table)


def _tc_body(sums_ref, texts_ref, wt_ref, b_ref, out_ref, mask_ref):
    cnt = jnp.sum((texts_ref[...] != 0).astype(jnp.float32), axis=1,
                  keepdims=True)
    s = cnt + _EPS
    avg = sums_ref[...] / s
    out_ref[...] = (
        jnp.dot(avg, wt_ref[...], preferred_element_type=jnp.float32)
        + b_ref[...]
    )
    mask_ref[...] = (s != 0).astype(jnp.int32)


def _tc_finish(sums, texts, wt, b2):
    bt = 2048
    grid = (_B // bt,)
    return pl.pallas_call(
        _tc_body,
        grid=grid,
        in_specs=[
            pl.BlockSpec((bt, _E), lambda i: (i, 0)),
            pl.BlockSpec((bt, _L), lambda i: (i, 0)),
            pl.BlockSpec((_E, _E), lambda i: (0, 0)),
            pl.BlockSpec((1, _E), lambda i: (0, 0)),
        ],
        out_specs=[
            pl.BlockSpec((bt, _E), lambda i: (i, 0)),
            pl.BlockSpec((bt, 1), lambda i: (i, 0)),
        ],
        out_shape=[
            jax.ShapeDtypeStruct((_B, _E), jnp.float32),
            jax.ShapeDtypeStruct((_B, 1), jnp.int32),
        ],
    )(sums, texts, wt, b2)


def kernel(texts, table, W, b):
    assert texts.shape == (_B, _L)
    texts2 = texts.reshape(_B // 2, 2 * _L)
    sums = _sc_gather_sum(texts2, table)
    out, mask = _tc_finish(sums, texts, W.T, b.reshape(1, _E))
    return (out, mask)
